# baseline (speedup 1.0000x reference)
import numpy as np
import jax
import jax.numpy as jnp
from jax import lax
from jax.experimental import pallas as pl
from jax.experimental.pallas import tpu as pltpu
from jax.experimental.pallas import tpu_sc as plsc

B = 4096
F = 26
VOCAB = 1000
D = 16
P = F * (F - 1) // 2
H1 = 1024
H2 = 512
EPS = 1e-5
SLAB = F * D
FP = 32
SLABP = FP * D

_IU, _JU = np.triu_indices(F, k=1)
_IU = [int(v) for v in _IU]
_JU = [int(v) for v in _JU]

NC, NS = 2, 16
NW = NC * NS
ROWS_W = B // NW
CH = 8
SUB = 2
NSUB = ROWS_W // SUB
NCHUNK = ROWS_W // CH
SUBW = SUB * F + 4


def _sc_gather_body(table_hbm, wlin_hbm, idx_hbm, em_hbm, lin_hbm,
                    idx_all, lin_all, slabs0, slabs1, em_v,
                    sem0, sem1, sem_w, sem_l):
    wid = lax.axis_index("s") * NC + lax.axis_index("c")
    row0 = wid * ROWS_W
    base_i = (row0 // SUB) * SUBW
    n_i = NSUB * SUBW
    pltpu.sync_copy(idx_hbm.at[pl.ds(base_i, n_i)], idx_all)
    lin_cps = [pltpu.make_async_copy(
        wlin_hbm.at[idx_all.at[pl.ds(k * 128, 128)]],
        lin_all.at[pl.ds(k * 128, 128)], sem_l)
        for k in range(n_i // 128)]
    for cp in lin_cps:
        cp.start()
    sems = (sem0, sem1)
    bufs = (slabs0, slabs1)
    cp0 = pltpu.make_async_copy(
        table_hbm.at[idx_all.at[pl.ds(0, SUBW)]], slabs0, sem0)
    cp0.start()
    cp0.wait()

    def compute_sub(buf, s_lo):
        def row_body(r, carry2):
            sb = r * F
            for p in range(P):
                i = _IU[p]
                j = _JU[p]
                a = buf[sb + i, pl.ds(j * D, D)]
                bv = buf[sb + j, pl.ds(i * D, D)]
                em_v[s_lo + r, pl.ds(p * D, D)] = a * bv
            return carry2
        lax.fori_loop(0, SUB, row_body, 0)

    def chunk_body(c, carry):
        base_b = row0 + c * CH

        @pl.when(c > 0)
        def _drain_prev_em():
            pltpu.make_async_copy(em_v, em_hbm.at[pl.ds(base_b, CH)],
                                  sem_w).wait()

        def pair_body(pp, carry2):
            for sub2 in range(2):
                l = c * (CH // SUB) + pp * 2 + sub2
                nxt_off = jnp.minimum(l + 1, NSUB - 1) * SUBW
                nxt = pltpu.make_async_copy(
                    table_hbm.at[idx_all.at[pl.ds(nxt_off, SUBW)]],
                    bufs[1 - sub2], sems[1 - sub2])

                @pl.when(l + 1 < NSUB)
                def _start_next():
                    nxt.start()

                compute_sub(bufs[sub2], (pp * 2 + sub2) * SUB)

                @pl.when(l + 1 < NSUB)
                def _wait_next():
                    nxt.wait()

            return carry2

        lax.fori_loop(0, (CH // SUB) // 2, pair_body, 0)
        pltpu.make_async_copy(em_v, em_hbm.at[pl.ds(base_b, CH)],
                              sem_w).start()
        return carry

    lax.fori_loop(0, NCHUNK, chunk_body, 0)
    pltpu.make_async_copy(
        em_v, em_hbm.at[pl.ds(row0 + (NCHUNK - 1) * CH, CH)], sem_w).wait()
    for cp in lin_cps:
        cp.wait()
    pltpu.sync_copy(lin_all, lin_hbm.at[pl.ds(base_i, n_i)])


import functools


@functools.cache
def _sc_gather_fn():
    return pl.kernel(
        _sc_gather_body,
        out_type=(jax.ShapeDtypeStruct((B, P * D), jnp.float32),
                  jax.ShapeDtypeStruct(((B // SUB) * SUBW,), jnp.float32)),
        mesh=plsc.VectorSubcoreMesh(core_axis_name="c", subcore_axis_name="s",
                                    num_cores=NC, num_subcores=NS),
        scratch_types=(pltpu.VMEM((NSUB * SUBW,), jnp.int32),
                       pltpu.VMEM((NSUB * SUBW,), jnp.float32),
                       pltpu.VMEM((SUBW, SLABP), jnp.float32),
                       pltpu.VMEM((SUBW, SLABP), jnp.float32),
                       pltpu.VMEM((CH, P * D), jnp.float32),
                       pltpu.SemaphoreType.DMA,
                       pltpu.SemaphoreType.DMA,
                       pltpu.SemaphoreType.DMA,
                       pltpu.SemaphoreType.DMA),
    )

BB = 512


def _l1_body(em_ref, w1_ref, h1_ref, s1_ref, q1_ref):
    h = jnp.dot(em_ref[...].astype(jnp.bfloat16), w1_ref[...],
                preferred_element_type=jnp.float32)
    h1_ref[...] = h

    @pl.when(pl.program_id(0) == 0)
    def _init():
        s1_ref[...] = jnp.zeros_like(s1_ref)
        q1_ref[...] = jnp.zeros_like(q1_ref)

    s1_ref[...] += jnp.sum(h, axis=0, keepdims=True)
    q1_ref[...] += jnp.sum(h * h, axis=0, keepdims=True)


_l1 = pl.pallas_call(
    _l1_body,
    grid=(B // BB,),
    in_specs=[pl.BlockSpec((BB, P * D), lambda i: (i, 0)),
              pl.BlockSpec((P * D, H1), lambda i: (0, 0))],
    out_specs=[pl.BlockSpec((BB, H1), lambda i: (i, 0)),
               pl.BlockSpec((1, H1), lambda i: (0, 0)),
               pl.BlockSpec((1, H1), lambda i: (0, 0))],
    out_shape=[jax.ShapeDtypeStruct((B, H1), jnp.float32),
               jax.ShapeDtypeStruct((1, H1), jnp.float32),
               jax.ShapeDtypeStruct((1, H1), jnp.float32)],
)


def _l2_body(h1_ref, s1_ref, q1_ref, g1_ref, bt1_ref, w2_ref,
             h2_ref, s2_ref, q2_ref):
    mu = s1_ref[...] * (1.0 / B)
    var = q1_ref[...] * (1.0 / B) - mu * mu
    scale = g1_ref[...] * lax.rsqrt(var + EPS)
    hn = jnp.maximum(h1_ref[...] * scale + (bt1_ref[...] - mu * scale), 0.0)
    h2 = jnp.dot(hn.astype(jnp.bfloat16), w2_ref[...],
                 preferred_element_type=jnp.float32)
    h2_ref[...] = h2

    @pl.when(pl.program_id(0) == 0)
    def _init():
        s2_ref[...] = jnp.zeros_like(s2_ref)
        q2_ref[...] = jnp.zeros_like(q2_ref)

    s2_ref[...] += jnp.sum(h2, axis=0, keepdims=True)
    q2_ref[...] += jnp.sum(h2 * h2, axis=0, keepdims=True)


_l2 = pl.pallas_call(
    _l2_body,
    grid=(B // BB,),
    in_specs=[pl.BlockSpec((BB, H1), lambda i: (i, 0)),
              pl.BlockSpec((1, H1), lambda i: (0, 0)),
              pl.BlockSpec((1, H1), lambda i: (0, 0)),
              pl.BlockSpec((1, H1), lambda i: (0, 0)),
              pl.BlockSpec((1, H1), lambda i: (0, 0)),
              pl.BlockSpec((H1, H2), lambda i: (0, 0))],
    out_specs=[pl.BlockSpec((BB, H2), lambda i: (i, 0)),
               pl.BlockSpec((1, H2), lambda i: (0, 0)),
               pl.BlockSpec((1, H2), lambda i: (0, 0))],
    out_shape=[jax.ShapeDtypeStruct((B, H2), jnp.float32),
               jax.ShapeDtypeStruct((1, H2), jnp.float32),
               jax.ShapeDtypeStruct((1, H2), jnp.float32)],
)


def _l3_body(h2_ref, s2_ref, q2_ref, g2_ref, bt2_ref, w3_ref, lin_ref, c_ref,
             out_ref):
    mu = s2_ref[...] * (1.0 / B)
    var = q2_ref[...] * (1.0 / B) - mu * mu
    scale = g2_ref[...] * lax.rsqrt(var + EPS)
    hn = jnp.maximum(h2_ref[...] * scale + (bt2_ref[...] - mu * scale), 0.0)
    y = jnp.sum(hn * w3_ref[...], axis=1, keepdims=True)
    ylin = jnp.sum(lin_ref[...], axis=1, keepdims=True)
    out_ref[...] = jax.nn.sigmoid(y + ylin + c_ref[...])


_l3 = pl.pallas_call(
    _l3_body,
    grid=(B // BB,),
    in_specs=[pl.BlockSpec((BB, H2), lambda i: (i, 0)),
              pl.BlockSpec((1, H2), lambda i: (0, 0)),
              pl.BlockSpec((1, H2), lambda i: (0, 0)),
              pl.BlockSpec((1, H2), lambda i: (0, 0)),
              pl.BlockSpec((1, H2), lambda i: (0, 0)),
              pl.BlockSpec((1, H2), lambda i: (0, 0)),
              pl.BlockSpec((BB, F), lambda i: (i, 0)),
              pl.BlockSpec((1, 1), lambda i: (0, 0))],
    out_specs=pl.BlockSpec((BB, 1), lambda i: (i, 0)),
    out_shape=jax.ShapeDtypeStruct((B, 1), jnp.float32),
)


def kernel(x, W_linear, ffm_tables, W1, b1, g1, beta1, W2, b2, g2, beta2,
           W3, b3, b):
    table = jnp.pad(ffm_tables.reshape(F * VOCAB, F, D),
                    ((0, 0), (0, FP - F), (0, 0))).reshape(F * VOCAB, SLABP)
    wlin = W_linear.reshape(F * VOCAB)
    idx = (x + jnp.arange(F, dtype=jnp.int32)[None, :] * VOCAB).reshape(-1)
    idxp = jnp.pad(idx.reshape(B // SUB, SUB * F), ((0, 0), (0, SUBW - SUB * F)),
                   mode="edge").reshape(-1)
    flat, linp = _sc_gather_fn()(table, wlin, idxp)
    lin = linp.reshape(B // SUB, SUBW)[:, :SUB * F]
    h1, s1, q1 = _l1(flat, W1.astype(jnp.bfloat16))
    h2, s2, q2 = _l2(h1, s1, q1, g1.reshape(1, H1), beta1.reshape(1, H1),
                     W2.astype(jnp.bfloat16))
    c = (b3 + b).reshape(1, 1)
    out = _l3(h2, s2, q2, g2.reshape(1, H2), beta2.reshape(1, H2),
              W3.reshape(1, H2), lin.reshape(B, F), c)
    return out.reshape(B)


# speedup vs baseline: 1.0390x; 1.0390x over previous
import numpy as np
import jax
import jax.numpy as jnp
from jax import lax
from jax.experimental import pallas as pl
from jax.experimental.pallas import tpu as pltpu
from jax.experimental.pallas import tpu_sc as plsc

B = 4096
F = 26
VOCAB = 1000
D = 16
P = F * (F - 1) // 2
H1 = 1024
H2 = 512
EPS = 1e-5
SLAB = F * D
FP = 32
SLABP = FP * D

_IU, _JU = np.triu_indices(F, k=1)
_IU = [int(v) for v in _IU]
_JU = [int(v) for v in _JU]

NC, NS = 2, 16
NW = NC * NS
ROWS_W = B // NW
CH = 8
SUB = 4
NSUB = ROWS_W // SUB
NCHUNK = ROWS_W // CH


def _sc_gather_body(table_hbm, wlin_hbm, idx_hbm, em_hbm, lin_hbm,
                    idx_all, lin_all, slabs_v, em_v,
                    sem0, sem_w, sem_l):
    wid = lax.axis_index("s") * NC + lax.axis_index("c")
    row0 = wid * ROWS_W
    base_i = row0 * F
    n_i = ROWS_W * F
    pltpu.sync_copy(idx_hbm.at[pl.ds(base_i, n_i)], idx_all)
    lin_cps = [pltpu.make_async_copy(
        wlin_hbm.at[idx_all.at[pl.ds(k * 128, 128)]],
        lin_all.at[pl.ds(k * 128, 128)], sem_l)
        for k in range(n_i // 128)]
    for cp in lin_cps:
        cp.start()

    def compute_sub(s_lo):
        def row_body(r, carry2):
            sb = r * F
            for p in range(P):
                i = _IU[p]
                j = _JU[p]
                a = slabs_v[sb + i, pl.ds(j * D, D)]
                bv = slabs_v[sb + j, pl.ds(i * D, D)]
                em_v[s_lo + r, pl.ds(p * D, D)] = a * bv
            return carry2
        lax.fori_loop(0, SUB, row_body, 0)

    def chunk_body(c, carry):
        base_b = row0 + c * CH

        @pl.when(c > 0)
        def _drain_prev_em():
            pltpu.make_async_copy(em_v, em_hbm.at[pl.ds(base_b, CH)],
                                  sem_w).wait()

        for half in range(CH // SUB):
            l = c * (CH // SUB) + half
            cp = pltpu.make_async_copy(
                table_hbm.at[idx_all.at[pl.ds(l * (SUB * F), SUB * F)]],
                slabs_v, sem0)
            cp.start()
            cp.wait()
            compute_sub(half * SUB)
        pltpu.make_async_copy(em_v, em_hbm.at[pl.ds(base_b, CH)],
                              sem_w).start()
        return carry

    lax.fori_loop(0, NCHUNK, chunk_body, 0)
    pltpu.make_async_copy(
        em_v, em_hbm.at[pl.ds(row0 + (NCHUNK - 1) * CH, CH)], sem_w).wait()
    for cp in lin_cps:
        cp.wait()
    pltpu.sync_copy(lin_all, lin_hbm.at[pl.ds(base_i, n_i)])


import functools


@functools.cache
def _sc_gather_fn():
    return pl.kernel(
        _sc_gather_body,
        out_type=(jax.ShapeDtypeStruct((B, P * D), jnp.float32),
                  jax.ShapeDtypeStruct((B * F,), jnp.float32)),
        mesh=plsc.VectorSubcoreMesh(core_axis_name="c", subcore_axis_name="s",
                                    num_cores=NC, num_subcores=NS),
        scratch_types=(pltpu.VMEM((ROWS_W * F,), jnp.int32),
                       pltpu.VMEM((ROWS_W * F,), jnp.float32),
                       pltpu.VMEM((SUB * F, SLABP), jnp.float32),
                       pltpu.VMEM((CH, P * D), jnp.float32),
                       pltpu.SemaphoreType.DMA,
                       pltpu.SemaphoreType.DMA,
                       pltpu.SemaphoreType.DMA),
    )


BB = 512


def _l1_body(em_ref, w1_ref, h1_ref, s1_ref, q1_ref):
    h = jnp.dot(em_ref[...].astype(jnp.bfloat16), w1_ref[...],
                preferred_element_type=jnp.float32)
    h1_ref[...] = h

    @pl.when(pl.program_id(0) == 0)
    def _init():
        s1_ref[...] = jnp.zeros_like(s1_ref)
        q1_ref[...] = jnp.zeros_like(q1_ref)

    s1_ref[...] += jnp.sum(h, axis=0, keepdims=True)
    q1_ref[...] += jnp.sum(h * h, axis=0, keepdims=True)


_l1 = pl.pallas_call(
    _l1_body,
    grid=(B // BB,),
    in_specs=[pl.BlockSpec((BB, P * D), lambda i: (i, 0)),
              pl.BlockSpec((P * D, H1), lambda i: (0, 0))],
    out_specs=[pl.BlockSpec((BB, H1), lambda i: (i, 0)),
               pl.BlockSpec((1, H1), lambda i: (0, 0)),
               pl.BlockSpec((1, H1), lambda i: (0, 0))],
    out_shape=[jax.ShapeDtypeStruct((B, H1), jnp.float32),
               jax.ShapeDtypeStruct((1, H1), jnp.float32),
               jax.ShapeDtypeStruct((1, H1), jnp.float32)],
)


def _l2_body(h1_ref, s1_ref, q1_ref, g1_ref, bt1_ref, w2_ref,
             h2_ref, s2_ref, q2_ref):
    mu = s1_ref[...] * (1.0 / B)
    var = q1_ref[...] * (1.0 / B) - mu * mu
    scale = g1_ref[...] * lax.rsqrt(var + EPS)
    hn = jnp.maximum(h1_ref[...] * scale + (bt1_ref[...] - mu * scale), 0.0)
    h2 = jnp.dot(hn.astype(jnp.bfloat16), w2_ref[...],
                 preferred_element_type=jnp.float32)
    h2_ref[...] = h2

    @pl.when(pl.program_id(0) == 0)
    def _init():
        s2_ref[...] = jnp.zeros_like(s2_ref)
        q2_ref[...] = jnp.zeros_like(q2_ref)

    s2_ref[...] += jnp.sum(h2, axis=0, keepdims=True)
    q2_ref[...] += jnp.sum(h2 * h2, axis=0, keepdims=True)


_l2 = pl.pallas_call(
    _l2_body,
    grid=(B // BB,),
    in_specs=[pl.BlockSpec((BB, H1), lambda i: (i, 0)),
              pl.BlockSpec((1, H1), lambda i: (0, 0)),
              pl.BlockSpec((1, H1), lambda i: (0, 0)),
              pl.BlockSpec((1, H1), lambda i: (0, 0)),
              pl.BlockSpec((1, H1), lambda i: (0, 0)),
              pl.BlockSpec((H1, H2), lambda i: (0, 0))],
    out_specs=[pl.BlockSpec((BB, H2), lambda i: (i, 0)),
               pl.BlockSpec((1, H2), lambda i: (0, 0)),
               pl.BlockSpec((1, H2), lambda i: (0, 0))],
    out_shape=[jax.ShapeDtypeStruct((B, H2), jnp.float32),
               jax.ShapeDtypeStruct((1, H2), jnp.float32),
               jax.ShapeDtypeStruct((1, H2), jnp.float32)],
)


def _l3_body(h2_ref, s2_ref, q2_ref, g2_ref, bt2_ref, w3_ref, lin_ref, c_ref,
             out_ref):
    mu = s2_ref[...] * (1.0 / B)
    var = q2_ref[...] * (1.0 / B) - mu * mu
    scale = g2_ref[...] * lax.rsqrt(var + EPS)
    hn = jnp.maximum(h2_ref[...] * scale + (bt2_ref[...] - mu * scale), 0.0)
    y = jnp.sum(hn * w3_ref[...], axis=1, keepdims=True)
    ylin = jnp.sum(lin_ref[...], axis=1, keepdims=True)
    out_ref[...] = jax.nn.sigmoid(y + ylin + c_ref[...])


_l3 = pl.pallas_call(
    _l3_body,
    grid=(B // BB,),
    in_specs=[pl.BlockSpec((BB, H2), lambda i: (i, 0)),
              pl.BlockSpec((1, H2), lambda i: (0, 0)),
              pl.BlockSpec((1, H2), lambda i: (0, 0)),
              pl.BlockSpec((1, H2), lambda i: (0, 0)),
              pl.BlockSpec((1, H2), lambda i: (0, 0)),
              pl.BlockSpec((1, H2), lambda i: (0, 0)),
              pl.BlockSpec((BB, F), lambda i: (i, 0)),
              pl.BlockSpec((1, 1), lambda i: (0, 0))],
    out_specs=pl.BlockSpec((BB, 1), lambda i: (i, 0)),
    out_shape=jax.ShapeDtypeStruct((B, 1), jnp.float32),
)


def kernel(x, W_linear, ffm_tables, W1, b1, g1, beta1, W2, b2, g2, beta2,
           W3, b3, b):
    table = jnp.pad(ffm_tables.reshape(F * VOCAB, F, D),
                    ((0, 0), (0, FP - F), (0, 0))).reshape(F * VOCAB, SLABP)
    wlin = W_linear.reshape(F * VOCAB)
    idx = (x + jnp.arange(F, dtype=jnp.int32)[None, :] * VOCAB).reshape(-1)
    flat, lin = _sc_gather_fn()(table, wlin, idx)
    h1, s1, q1 = _l1(flat, W1.astype(jnp.bfloat16))
    h2, s2, q2 = _l2(h1, s1, q1, g1.reshape(1, H1), beta1.reshape(1, H1),
                     W2.astype(jnp.bfloat16))
    c = (b3 + b).reshape(1, 1)
    out = _l3(h2, s2, q2, g2.reshape(1, H2), beta2.reshape(1, H2),
              W3.reshape(1, H2), lin.reshape(B, F), c)
    return out.reshape(B)


# speedup vs baseline: 1.1637x; 1.1201x over previous
import numpy as np
import jax
import jax.numpy as jnp
from jax import lax
from jax.experimental import pallas as pl
from jax.experimental.pallas import tpu as pltpu
from jax.experimental.pallas import tpu_sc as plsc

B = 4096
F = 26
VOCAB = 1000
D = 16
P = F * (F - 1) // 2
H1 = 1024
H2 = 512
EPS = 1e-5
SLAB = F * D
FP = 32
SLABP = FP * D

_IU, _JU = np.triu_indices(F, k=1)
_IU = [int(v) for v in _IU]
_JU = [int(v) for v in _JU]

NC, NS = 2, 16
NW = NC * NS
ROWS_W = B // NW
CH = 8
SUB = 4
NSUB = ROWS_W // SUB
NCHUNK = ROWS_W // CH


def _sc_gather_body(table_hbm, wlin_hbm, idx_hbm, em_hbm, lin_hbm,
                    idx_all, lin_all, slabs_v, em_v,
                    sem0, sem_w, sem_l):
    wid = lax.axis_index("s") * NC + lax.axis_index("c")
    row0 = wid * ROWS_W
    base_i = row0 * F
    n_i = ROWS_W * F
    pltpu.sync_copy(idx_hbm.at[pl.ds(base_i, n_i)], idx_all)
    lin_cps = [pltpu.make_async_copy(
        wlin_hbm.at[idx_all.at[pl.ds(k * 128, 128)]],
        lin_all.at[pl.ds(k * 128, 128)], sem_l)
        for k in range(n_i // 128)]
    for cp in lin_cps:
        cp.start()

    def compute_sub(s_lo):
        def row_body(r, carry2):
            sb = r * F
            for p in range(P):
                i = _IU[p]
                j = _JU[p]
                a = slabs_v[sb + i, pl.ds(j * D, D)]
                bv = slabs_v[sb + j, pl.ds(i * D, D)]
                em_v[s_lo + r, pl.ds(p * D, D)] = a * bv
            return carry2
        lax.fori_loop(0, SUB, row_body, 0)

    def chunk_body(c, carry):
        base_b = row0 + c * CH

        @pl.when(c > 0)
        def _drain_prev_em():
            pltpu.make_async_copy(em_v, em_hbm.at[pl.ds(base_b, CH)],
                                  sem_w).wait()

        for half in range(CH // SUB):
            l = c * (CH // SUB) + half
            cp = pltpu.make_async_copy(
                table_hbm.at[idx_all.at[pl.ds(l * (SUB * F), SUB * F)]],
                slabs_v, sem0)
            cp.start()
            cp.wait()
            compute_sub(half * SUB)
        pltpu.make_async_copy(em_v, em_hbm.at[pl.ds(base_b, CH)],
                              sem_w).start()
        return carry

    lax.fori_loop(0, NCHUNK, chunk_body, 0)
    pltpu.make_async_copy(
        em_v, em_hbm.at[pl.ds(row0 + (NCHUNK - 1) * CH, CH)], sem_w).wait()
    for cp in lin_cps:
        cp.wait()
    pltpu.sync_copy(lin_all, lin_hbm.at[pl.ds(base_i, n_i)])


import functools


@functools.cache
def _sc_gather_fn():
    return pl.kernel(
        _sc_gather_body,
        out_type=(jax.ShapeDtypeStruct((B, P * D), jnp.float32),
                  jax.ShapeDtypeStruct((B * F,), jnp.float32)),
        mesh=plsc.VectorSubcoreMesh(core_axis_name="c", subcore_axis_name="s",
                                    num_cores=NC, num_subcores=NS),
        scratch_types=(pltpu.VMEM((ROWS_W * F,), jnp.int32),
                       pltpu.VMEM((ROWS_W * F,), jnp.float32),
                       pltpu.VMEM((SUB * F, SLABP), jnp.float32),
                       pltpu.VMEM((CH, P * D), jnp.float32),
                       pltpu.SemaphoreType.DMA,
                       pltpu.SemaphoreType.DMA,
                       pltpu.SemaphoreType.DMA),
    )


BB = 512


def _l1_body(em_ref, w1_ref, h1_ref, s1_ref, q1_ref):
    h = jnp.dot(em_ref[...].astype(jnp.bfloat16), w1_ref[...],
                preferred_element_type=jnp.float32)
    h1_ref[...] = h

    @pl.when(pl.program_id(0) == 0)
    def _init():
        s1_ref[...] = jnp.zeros_like(s1_ref)
        q1_ref[...] = jnp.zeros_like(q1_ref)

    s1_ref[...] += jnp.sum(h, axis=0, keepdims=True)
    q1_ref[...] += jnp.sum(h * h, axis=0, keepdims=True)


_l1 = pl.pallas_call(
    _l1_body,
    grid=(B // BB,),
    in_specs=[pl.BlockSpec((BB, P * D), lambda i: (i, 0)),
              pl.BlockSpec((P * D, H1), lambda i: (0, 0))],
    out_specs=[pl.BlockSpec((BB, H1), lambda i: (i, 0)),
               pl.BlockSpec((1, H1), lambda i: (0, 0)),
               pl.BlockSpec((1, H1), lambda i: (0, 0))],
    out_shape=[jax.ShapeDtypeStruct((B, H1), jnp.float32),
               jax.ShapeDtypeStruct((1, H1), jnp.float32),
               jax.ShapeDtypeStruct((1, H1), jnp.float32)],
)


def _l2_body(h1_ref, s1_ref, q1_ref, g1_ref, bt1_ref, w2_ref,
             h2_ref, s2_ref, q2_ref):
    mu = s1_ref[...] * (1.0 / B)
    var = q1_ref[...] * (1.0 / B) - mu * mu
    scale = g1_ref[...] * lax.rsqrt(var + EPS)
    hn = jnp.maximum(h1_ref[...] * scale + (bt1_ref[...] - mu * scale), 0.0)
    h2 = jnp.dot(hn.astype(jnp.bfloat16), w2_ref[...],
                 preferred_element_type=jnp.float32)
    h2_ref[...] = h2

    @pl.when(pl.program_id(0) == 0)
    def _init():
        s2_ref[...] = jnp.zeros_like(s2_ref)
        q2_ref[...] = jnp.zeros_like(q2_ref)

    s2_ref[...] += jnp.sum(h2, axis=0, keepdims=True)
    q2_ref[...] += jnp.sum(h2 * h2, axis=0, keepdims=True)


_l2 = pl.pallas_call(
    _l2_body,
    grid=(B // BB,),
    in_specs=[pl.BlockSpec((BB, H1), lambda i: (i, 0)),
              pl.BlockSpec((1, H1), lambda i: (0, 0)),
              pl.BlockSpec((1, H1), lambda i: (0, 0)),
              pl.BlockSpec((1, H1), lambda i: (0, 0)),
              pl.BlockSpec((1, H1), lambda i: (0, 0)),
              pl.BlockSpec((H1, H2), lambda i: (0, 0))],
    out_specs=[pl.BlockSpec((BB, H2), lambda i: (i, 0)),
               pl.BlockSpec((1, H2), lambda i: (0, 0)),
               pl.BlockSpec((1, H2), lambda i: (0, 0))],
    out_shape=[jax.ShapeDtypeStruct((B, H2), jnp.float32),
               jax.ShapeDtypeStruct((1, H2), jnp.float32),
               jax.ShapeDtypeStruct((1, H2), jnp.float32)],
)


def _l3_body(h2_ref, s2_ref, q2_ref, g2_ref, bt2_ref, w3_ref, lin_ref, c_ref,
             out_ref):
    mu = s2_ref[...] * (1.0 / B)
    var = q2_ref[...] * (1.0 / B) - mu * mu
    scale = g2_ref[...] * lax.rsqrt(var + EPS)
    hn = jnp.maximum(h2_ref[...] * scale + (bt2_ref[...] - mu * scale), 0.0)
    y = jnp.sum(hn * w3_ref[...], axis=1, keepdims=True)
    ylin = jnp.sum(lin_ref[...], axis=1, keepdims=True)
    out_ref[...] = jax.nn.sigmoid(y + ylin + c_ref[...])


_l3 = pl.pallas_call(
    _l3_body,
    grid=(B // BB,),
    in_specs=[pl.BlockSpec((BB, H2), lambda i: (i, 0)),
              pl.BlockSpec((1, H2), lambda i: (0, 0)),
              pl.BlockSpec((1, H2), lambda i: (0, 0)),
              pl.BlockSpec((1, H2), lambda i: (0, 0)),
              pl.BlockSpec((1, H2), lambda i: (0, 0)),
              pl.BlockSpec((1, H2), lambda i: (0, 0)),
              pl.BlockSpec((BB, F), lambda i: (i, 0)),
              pl.BlockSpec((1, 1), lambda i: (0, 0))],
    out_specs=pl.BlockSpec((BB, 1), lambda i: (i, 0)),
    out_shape=jax.ShapeDtypeStruct((B, 1), jnp.float32),
)


def kernel(x, W_linear, ffm_tables, W1, b1, g1, beta1, W2, b2, g2, beta2,
           W3, b3, b):
    table = jnp.concatenate(
        [ffm_tables.reshape(F * VOCAB, SLAB),
         jnp.zeros((F * VOCAB, SLABP - SLAB), jnp.float32)], axis=1)
    wlin = W_linear.reshape(F * VOCAB)
    idx = (x + jnp.arange(F, dtype=jnp.int32)[None, :] * VOCAB).reshape(-1)
    flat, lin = _sc_gather_fn()(table, wlin, idx)
    h1, s1, q1 = _l1(flat, W1.astype(jnp.bfloat16))
    h2, s2, q2 = _l2(h1, s1, q1, g1.reshape(1, H1), beta1.reshape(1, H1),
                     W2.astype(jnp.bfloat16))
    c = (b3 + b).reshape(1, 1)
    out = _l3(h2, s2, q2, g2.reshape(1, H2), beta2.reshape(1, H2),
              W3.reshape(1, H2), lin.reshape(B, F), c)
    return out.reshape(B)
